# baseline (speedup 1.0000x reference)
import math

import jax
import jax.numpy as jnp
from jax import lax
from jax.experimental import pallas as pl
from jax.experimental.pallas import tpu as pltpu
from jax.experimental.pallas import tpu_sc as plsc

_EMBED = 64
_ROWS = 16384
_COLS = 50
_TABLE = 360

_NC = 2
_NS = 16
_NW = _NC * _NS
_RPW = _ROWS // _NW


def _sc_body(ang_hbm, table_hbm, out_hbm, ang_v, table_v, pan0, pan1,
             wsem0, wsem1):
    wid = lax.axis_index("s") * _NC + lax.axis_index("c")
    b0 = wid * _RPW
    pans = (pan0, pan1)
    wsems = (wsem0, wsem1)

    pltpu.sync_copy(ang_hbm.at[:, pl.ds(b0, _RPW)], ang_v)
    pltpu.sync_copy(table_hbm, table_v)

    def fill(c, pan):
        def lg_body(t, carry):
            sl = pl.ds(t * 16, 16)
            base = ang_v[c, sl].astype(jnp.int32) * _EMBED
            for e in range(_EMBED):
                g = plsc.load_gather(table_v, [base + e])
                pan[e, sl] = g
            return carry

        lax.fori_loop(0, _RPW // 16, lg_body, 0)

    def wdesc(c, b):
        return pltpu.make_async_copy(
            pans[b], out_hbm.at[c, :, pl.ds(b0, _RPW)], wsems[b])

    fill(0, pan0)
    wdesc(0, 0).start()
    fill(1, pan1)
    wdesc(1, 1).start()

    def body(p, carry):
        c0 = 2 * p + 2
        wdesc(c0 - 2, 0).wait()
        fill(c0, pan0)
        wdesc(c0, 0).start()
        c1 = c0 + 1
        wdesc(c1 - 2, 1).wait()
        fill(c1, pan1)
        wdesc(c1, 1).start()
        return carry

    lax.fori_loop(0, (_COLS - 2) // 2, body, 0)
    wdesc(_COLS - 2, 0).wait()
    wdesc(_COLS - 1, 1).wait()


_sc_gather = pl.kernel(
    _sc_body,
    out_type=jax.ShapeDtypeStruct((_COLS, _EMBED, _ROWS), jnp.float32),
    mesh=plsc.VectorSubcoreMesh(core_axis_name="c", subcore_axis_name="s"),
    scratch_types=[
        pltpu.VMEM((_COLS, _RPW), jnp.float32),
        pltpu.VMEM((_TABLE * _EMBED,), jnp.float32),
        pltpu.VMEM((_EMBED, _RPW), jnp.float32),
        pltpu.VMEM((_EMBED, _RPW), jnp.float32),
        pltpu.SemaphoreType.DMA,
        pltpu.SemaphoreType.DMA,
    ],
    compiler_params=pltpu.CompilerParams(use_tc_tiling_on_sc=True,
                                         needs_layout_passes=False),
)


def _tc_trig_body(a_ref, rad_ref, sin_ref, cos_ref):
    r = a_ref[...] * jnp.float32(math.pi / 180.0)
    rad_ref[...] = r
    sin_ref[...] = jnp.sin(r)
    cos_ref[...] = jnp.cos(r)


_TC_BLOCK = 2048

_tc_trig = pl.pallas_call(
    _tc_trig_body,
    grid=(_ROWS // _TC_BLOCK,),
    in_specs=[pl.BlockSpec((_COLS, _TC_BLOCK), lambda i: (0, i))],
    out_specs=[pl.BlockSpec((_COLS, _TC_BLOCK), lambda i: (0, i))] * 3,
    out_shape=[jax.ShapeDtypeStruct((_COLS, _ROWS), jnp.float32)] * 3,
)


def kernel(angles, table):
    ang_t = angles.T
    rad_t, sin_t, cos_t = _tc_trig(ang_t)
    out_t = _sc_gather(ang_t, table.reshape(-1))
    return (rad_t.T, sin_t.T, cos_t.T,
            jnp.transpose(out_t, (2, 0, 1)))


# speedup vs baseline: 2.6111x; 2.6111x over previous
import math

import jax
import jax.numpy as jnp
from jax import lax
from jax.experimental import pallas as pl
from jax.experimental.pallas import tpu as pltpu
from jax.experimental.pallas import tpu_sc as plsc

_EMBED = 64
_ROWS = 16384
_COLS = 50
_TABLE = 360
_TSTRIDE = 65

_NC = 2
_NS = 16
_NW = _NC * _NS
_RPW = _ROWS // _NW


def _sc_body(ang_hbm, table_hbm, out_hbm, ang_v, table_v, pan0, pan1,
             wsem0, wsem1):
    wid = lax.axis_index("s") * _NC + lax.axis_index("c")
    b0 = wid * _RPW
    pans = (pan0, pan1)
    wsems = (wsem0, wsem1)

    pltpu.sync_copy(ang_hbm.at[:, pl.ds(b0, _RPW)], ang_v)
    pltpu.sync_copy(table_hbm, table_v)

    def fill(c, pan):
        def lg_body(t, carry):
            sls = [pl.ds((4 * t + j) * 16, 16) for j in range(4)]
            bases = [ang_v[c, sl].astype(jnp.int32) * _TSTRIDE for sl in sls]
            for e in range(_EMBED):
                for j in range(4):
                    g = plsc.load_gather(table_v, [bases[j] + e])
                    pan[e, sls[j]] = g
            return carry

        lax.fori_loop(0, _RPW // 64, lg_body, 0)

    def wdesc(c, b):
        return pltpu.make_async_copy(
            pans[b], out_hbm.at[c, :, pl.ds(b0, _RPW)], wsems[b])

    fill(0, pan0)
    wdesc(0, 0).start()
    fill(1, pan1)
    wdesc(1, 1).start()

    def body(p, carry):
        c0 = 2 * p + 2
        wdesc(c0 - 2, 0).wait()
        fill(c0, pan0)
        wdesc(c0, 0).start()
        c1 = c0 + 1
        wdesc(c1 - 2, 1).wait()
        fill(c1, pan1)
        wdesc(c1, 1).start()
        return carry

    lax.fori_loop(0, (_COLS - 2) // 2, body, 0)
    wdesc(_COLS - 2, 0).wait()
    wdesc(_COLS - 1, 1).wait()


_sc_gather = pl.kernel(
    _sc_body,
    out_type=jax.ShapeDtypeStruct((_COLS, _EMBED, _ROWS), jnp.float32),
    mesh=plsc.VectorSubcoreMesh(core_axis_name="c", subcore_axis_name="s"),
    scratch_types=[
        pltpu.VMEM((_COLS, _RPW), jnp.float32),
        pltpu.VMEM((_TABLE * _TSTRIDE,), jnp.float32),
        pltpu.VMEM((_EMBED, _RPW), jnp.float32),
        pltpu.VMEM((_EMBED, _RPW), jnp.float32),
        pltpu.SemaphoreType.DMA,
        pltpu.SemaphoreType.DMA,
    ],
    compiler_params=pltpu.CompilerParams(use_tc_tiling_on_sc=True,
                                         needs_layout_passes=False),
)


def _tc_trig_body(a_ref, rad_ref, sin_ref, cos_ref):
    r = a_ref[...] * jnp.float32(math.pi / 180.0)
    rad_ref[...] = r
    sin_ref[...] = jnp.sin(r)
    cos_ref[...] = jnp.cos(r)


_TC_BLOCK = 2048

_tc_trig = pl.pallas_call(
    _tc_trig_body,
    grid=(_ROWS // _TC_BLOCK,),
    in_specs=[pl.BlockSpec((_COLS, _TC_BLOCK), lambda i: (0, i))],
    out_specs=[pl.BlockSpec((_COLS, _TC_BLOCK), lambda i: (0, i))] * 3,
    out_shape=[jax.ShapeDtypeStruct((_COLS, _ROWS), jnp.float32)] * 3,
)


def kernel(angles, table):
    ang_t = angles.T
    rad_t, sin_t, cos_t = _tc_trig(ang_t)
    table_pad = jnp.pad(table, ((0, 0), (0, _TSTRIDE - _EMBED))).reshape(-1)
    out_t = _sc_gather(ang_t, table_pad)
    return (rad_t.T, sin_t.T, cos_t.T,
            jnp.transpose(out_t, (2, 0, 1)))


# speedup vs baseline: 7.6999x; 2.9489x over previous
import math

import jax
import jax.numpy as jnp
from jax import lax
from jax.experimental import pallas as pl
from jax.experimental.pallas import tpu as pltpu
from jax.experimental.pallas import tpu_sc as plsc

_EMBED = 64
_ROWS = 16384
_COLS = 50
_TABLE = 360
_TSTRIDE = 65

_NC = 2
_NS = 16
_NW = _NC * _NS
_RPW = _ROWS // _NW


def _sc_body(ang_hbm, table_hbm, out_hbm, ang_v, table_v, pan0, pan1,
             wsem0, wsem1):
    wid = lax.axis_index("s") * _NC + lax.axis_index("c")
    b0 = wid * _RPW
    pans = (pan0, pan1)
    wsems = (wsem0, wsem1)

    pltpu.sync_copy(ang_hbm.at[:, pl.ds(b0, _RPW)], ang_v)
    pltpu.sync_copy(table_hbm, table_v)

    def fill(c, pan):
        def lg_body(t, carry):
            sls = [pl.ds((4 * t + j) * 16, 16) for j in range(4)]
            bases = [ang_v[c, sl].astype(jnp.int32) * _TSTRIDE for sl in sls]
            for e in range(0, _EMBED, 2):
                gs = [(e + de, j,
                       plsc.load_gather(table_v, [bases[j] + (e + de)]))
                      for de in (0, 1) for j in range(4)]
                for ee, j, g in gs:
                    pan[ee, sls[j]] = g
            return carry

        lax.fori_loop(0, _RPW // 64, lg_body, 0)

    def wdesc(c, b):
        return pltpu.make_async_copy(
            pans[b], out_hbm.at[c, :, pl.ds(b0, _RPW)], wsems[b])

    fill(0, pan0)
    wdesc(0, 0).start()
    fill(1, pan1)
    wdesc(1, 1).start()

    def body(p, carry):
        c0 = 2 * p + 2
        wdesc(c0 - 2, 0).wait()
        fill(c0, pan0)
        wdesc(c0, 0).start()
        c1 = c0 + 1
        wdesc(c1 - 2, 1).wait()
        fill(c1, pan1)
        wdesc(c1, 1).start()
        return carry

    lax.fori_loop(0, (_COLS - 2) // 2, body, 0)
    wdesc(_COLS - 2, 0).wait()
    wdesc(_COLS - 1, 1).wait()


_sc_gather = pl.kernel(
    _sc_body,
    out_type=jax.ShapeDtypeStruct((_COLS, _EMBED, _ROWS), jnp.float32),
    mesh=plsc.VectorSubcoreMesh(core_axis_name="c", subcore_axis_name="s"),
    scratch_types=[
        pltpu.VMEM((_COLS, _RPW), jnp.float32),
        pltpu.VMEM((_TABLE * _TSTRIDE,), jnp.float32),
        pltpu.VMEM((_EMBED, _RPW), jnp.float32),
        pltpu.VMEM((_EMBED, _RPW), jnp.float32),
        pltpu.SemaphoreType.DMA,
        pltpu.SemaphoreType.DMA,
    ],
    compiler_params=pltpu.CompilerParams(use_tc_tiling_on_sc=True,
                                         needs_layout_passes=False),
)


def _tc_trig_body(a_ref, rad_ref, sin_ref, cos_ref):
    r = a_ref[...] * jnp.float32(math.pi / 180.0)
    rad_ref[...] = r
    sin_ref[...] = jnp.sin(r)
    cos_ref[...] = jnp.cos(r)


_TC_BLOCK = 2048

_tc_trig = pl.pallas_call(
    _tc_trig_body,
    grid=(_ROWS // _TC_BLOCK,),
    in_specs=[pl.BlockSpec((_COLS, _TC_BLOCK), lambda i: (0, i))],
    out_specs=[pl.BlockSpec((_COLS, _TC_BLOCK), lambda i: (0, i))] * 3,
    out_shape=[jax.ShapeDtypeStruct((_COLS, _ROWS), jnp.float32)] * 3,
)


def kernel(angles, table):
    ang_t = angles.T
    rad_t, sin_t, cos_t = _tc_trig(ang_t)
    table_pad = jnp.pad(table, ((0, 0), (0, _TSTRIDE - _EMBED))).reshape(-1)
    out_t = _sc_gather(ang_t, table_pad)
    return (rad_t.T, sin_t.T, cos_t.T,
            jnp.transpose(out_t, (2, 0, 1)))


# speedup vs baseline: 10.7466x; 1.3957x over previous
import math

import jax
import jax.numpy as jnp
from jax import lax
from jax.experimental import pallas as pl
from jax.experimental.pallas import tpu as pltpu
from jax.experimental.pallas import tpu_sc as plsc

_EMBED = 64
_ROWS = 16384
_COLS = 50
_TABLE = 360
_TSTRIDE = 65

_NC = 2
_NS = 16
_NW = _NC * _NS
_RPW = _ROWS // _NW


def _sc_body(ang_hbm, table_hbm, out_hbm, ang_v, table_v, pan0, pan1,
             wsem0, wsem1):
    wid = lax.axis_index("s") * _NC + lax.axis_index("c")
    b0 = wid * _RPW
    pans = (pan0, pan1)
    wsems = (wsem0, wsem1)

    pltpu.sync_copy(ang_hbm.at[:, pl.ds(b0, _RPW)], ang_v)
    pltpu.sync_copy(table_hbm, table_v)

    def fill(c, pan):
        def lg_body(t, carry):
            sls = [pl.ds((4 * t + j) * 16, 16) for j in range(4)]
            bases = [ang_v[c, sl].astype(jnp.int32) * _TSTRIDE for sl in sls]
            prev = None
            for e in range(0, _EMBED, 2):
                cur = []
                for i, (de, j) in enumerate([(d, j) for d in (0, 1)
                                             for j in range(4)]):
                    cur.append((e + de, j,
                                plsc.load_gather(table_v,
                                                 [bases[j] + (e + de)])))
                    if prev is not None:
                        ee, jj, g = prev[i]
                        pan[ee, sls[jj]] = g
                prev = cur
            for ee, jj, g in prev:
                pan[ee, sls[jj]] = g
            return carry

        lax.fori_loop(0, _RPW // 64, lg_body, 0)

    def wdesc(c, b):
        return pltpu.make_async_copy(
            pans[b], out_hbm.at[c, :, pl.ds(b0, _RPW)], wsems[b])

    fill(0, pan0)
    wdesc(0, 0).start()
    fill(1, pan1)
    wdesc(1, 1).start()

    def body(p, carry):
        c0 = 2 * p + 2
        wdesc(c0 - 2, 0).wait()
        fill(c0, pan0)
        wdesc(c0, 0).start()
        c1 = c0 + 1
        wdesc(c1 - 2, 1).wait()
        fill(c1, pan1)
        wdesc(c1, 1).start()
        return carry

    lax.fori_loop(0, (_COLS - 2) // 2, body, 0)
    wdesc(_COLS - 2, 0).wait()
    wdesc(_COLS - 1, 1).wait()


_sc_gather = pl.kernel(
    _sc_body,
    out_type=jax.ShapeDtypeStruct((_COLS, _EMBED, _ROWS), jnp.float32),
    mesh=plsc.VectorSubcoreMesh(core_axis_name="c", subcore_axis_name="s"),
    scratch_types=[
        pltpu.VMEM((_COLS, _RPW), jnp.float32),
        pltpu.VMEM((_TABLE * _TSTRIDE,), jnp.float32),
        pltpu.VMEM((_EMBED, _RPW), jnp.float32),
        pltpu.VMEM((_EMBED, _RPW), jnp.float32),
        pltpu.SemaphoreType.DMA,
        pltpu.SemaphoreType.DMA,
    ],
    compiler_params=pltpu.CompilerParams(use_tc_tiling_on_sc=True,
                                         needs_layout_passes=False),
)


def _tc_trig_body(a_ref, rad_ref, sin_ref, cos_ref):
    r = a_ref[...] * jnp.float32(math.pi / 180.0)
    rad_ref[...] = r
    sin_ref[...] = jnp.sin(r)
    cos_ref[...] = jnp.cos(r)


_TC_BLOCK = 2048

_tc_trig = pl.pallas_call(
    _tc_trig_body,
    grid=(_ROWS // _TC_BLOCK,),
    in_specs=[pl.BlockSpec((_COLS, _TC_BLOCK), lambda i: (0, i))],
    out_specs=[pl.BlockSpec((_COLS, _TC_BLOCK), lambda i: (0, i))] * 3,
    out_shape=[jax.ShapeDtypeStruct((_COLS, _ROWS), jnp.float32)] * 3,
)


def kernel(angles, table):
    ang_t = angles.T
    rad_t, sin_t, cos_t = _tc_trig(ang_t)
    table_pad = jnp.pad(table, ((0, 0), (0, _TSTRIDE - _EMBED))).reshape(-1)
    out_t = _sc_gather(ang_t, table_pad)
    return (rad_t.T, sin_t.T, cos_t.T,
            jnp.transpose(out_t, (2, 0, 1)))


# speedup vs baseline: 10.9045x; 1.0147x over previous
import math

import jax
import jax.numpy as jnp
from jax import lax
from jax.experimental import pallas as pl
from jax.experimental.pallas import tpu as pltpu
from jax.experimental.pallas import tpu_sc as plsc

_EMBED = 64
_ROWS = 16384
_COLS = 50
_TABLE = 360
_TSTRIDE = 65

_NC = 2
_NS = 16
_NW = _NC * _NS
_RPW = _ROWS // _NW


def _sc_body(ang_hbm, table_hbm, out_hbm, ang_v, table_v, pan0, pan1,
             wsem0, wsem1):
    wid = lax.axis_index("s") * _NC + lax.axis_index("c")
    b0 = wid * _RPW
    pans = (pan0, pan1)
    wsems = (wsem0, wsem1)

    stage_a = pltpu.make_async_copy(ang_hbm.at[:, pl.ds(b0, _RPW)], ang_v,
                                    wsem0)
    stage_t = pltpu.make_async_copy(table_hbm, table_v, wsem1)
    stage_a.start()
    stage_t.start()
    stage_a.wait()
    stage_t.wait()

    def fill(c, pan):
        def lg_body(t, carry):
            sls = [pl.ds((4 * t + j) * 16, 16) for j in range(4)]
            bases = [ang_v[c, sl].astype(jnp.int32) * _TSTRIDE for sl in sls]
            prev = None
            for e in range(0, _EMBED, 2):
                cur = []
                for i, (de, j) in enumerate([(d, j) for d in (0, 1)
                                             for j in range(4)]):
                    cur.append((e + de, j,
                                plsc.load_gather(table_v,
                                                 [bases[j] + (e + de)])))
                    if prev is not None:
                        ee, jj, g = prev[i]
                        pan[ee, sls[jj]] = g
                prev = cur
            for ee, jj, g in prev:
                pan[ee, sls[jj]] = g
            return carry

        lax.fori_loop(0, _RPW // 64, lg_body, 0)

    def wdesc(c, b):
        return pltpu.make_async_copy(
            pans[b], out_hbm.at[c, :, pl.ds(b0, _RPW)], wsems[b])

    fill(0, pan0)
    wdesc(0, 0).start()
    fill(1, pan1)
    wdesc(1, 1).start()

    def body(p, carry):
        c0 = 2 * p + 2
        wdesc(c0 - 2, 0).wait()
        fill(c0, pan0)
        wdesc(c0, 0).start()
        c1 = c0 + 1
        wdesc(c1 - 2, 1).wait()
        fill(c1, pan1)
        wdesc(c1, 1).start()
        return carry

    lax.fori_loop(0, (_COLS - 2) // 2, body, 0)
    wdesc(_COLS - 2, 0).wait()
    wdesc(_COLS - 1, 1).wait()


_sc_gather = pl.kernel(
    _sc_body,
    out_type=jax.ShapeDtypeStruct((_COLS, _EMBED, _ROWS), jnp.float32),
    mesh=plsc.VectorSubcoreMesh(core_axis_name="c", subcore_axis_name="s"),
    scratch_types=[
        pltpu.VMEM((_COLS, _RPW), jnp.float32),
        pltpu.VMEM((_TABLE * _TSTRIDE,), jnp.float32),
        pltpu.VMEM((_EMBED, _RPW), jnp.float32),
        pltpu.VMEM((_EMBED, _RPW), jnp.float32),
        pltpu.SemaphoreType.DMA,
        pltpu.SemaphoreType.DMA,
    ],
    compiler_params=pltpu.CompilerParams(use_tc_tiling_on_sc=True,
                                         needs_layout_passes=False),
)


def _tc_trig_body(a_ref, rad_ref, sin_ref, cos_ref):
    r = a_ref[...] * jnp.float32(math.pi / 180.0)
    rad_ref[...] = r
    sin_ref[...] = jnp.sin(r)
    cos_ref[...] = jnp.cos(r)


_TC_BLOCK = 2048

_tc_trig = pl.pallas_call(
    _tc_trig_body,
    grid=(_ROWS // _TC_BLOCK,),
    in_specs=[pl.BlockSpec((_COLS, _TC_BLOCK), lambda i: (0, i))],
    out_specs=[pl.BlockSpec((_COLS, _TC_BLOCK), lambda i: (0, i))] * 3,
    out_shape=[jax.ShapeDtypeStruct((_COLS, _ROWS), jnp.float32)] * 3,
)


def kernel(angles, table):
    ang_t = angles.T
    rad_t, sin_t, cos_t = _tc_trig(ang_t)
    table_pad = jnp.pad(table, ((0, 0), (0, _TSTRIDE - _EMBED))).reshape(-1)
    out_t = _sc_gather(ang_t, table_pad)
    return (rad_t.T, sin_t.T, cos_t.T,
            jnp.transpose(out_t, (2, 0, 1)))
